# baseline (device time: 34470 ns/iter reference)
import jax
import jax.numpy as jnp
from jax import lax
from jax.experimental import pallas as pl
from jax.experimental.pallas import tpu as pltpu


def kernel(x, dy):
    m, d = x.shape
    _, f = dy.shape
    fh = f // 2
    dh = d // 2

    def body(x_ref, dy_ref, out_ref, p_ref, r_ref, xrecv_ref, yrecv_ref,
             send_sems, recv_sems):
        px = lax.axis_index("x")
        py = lax.axis_index("y")

        barrier = pltpu.get_barrier_semaphore()
        pl.semaphore_signal(barrier, inc=1, device_id=(1 - px, py),
                            device_id_type=pl.DeviceIdType.MESH)
        pl.semaphore_signal(barrier, inc=1, device_id=(px, 1 - py),
                            device_id_type=pl.DeviceIdType.MESH)
        pl.semaphore_wait(barrier, 2)

        tdims = (((0,), (0,)), ((), ()))

        @pl.when(py == 0)
        def _():
            p_ref[...] = lax.dot_general(
                x_ref[...], dy_ref[:, :fh], tdims,
                preferred_element_type=jnp.float32)

        @pl.when(py == 1)
        def _():
            p_ref[...] = lax.dot_general(
                x_ref[...], dy_ref[:, fh:], tdims,
                preferred_element_type=jnp.float32)

        x_rdma = pltpu.make_async_remote_copy(
            src_ref=p_ref.at[pl.ds((1 - px) * dh, dh), :],
            dst_ref=xrecv_ref,
            send_sem=send_sems.at[0],
            recv_sem=recv_sems.at[0],
            device_id=(1 - px, py),
            device_id_type=pl.DeviceIdType.MESH,
        )
        x_rdma.start()
        x_rdma.wait()

        r_ref[...] = p_ref[pl.ds(px * dh, dh), :] + xrecv_ref[...]

        y_rdma = pltpu.make_async_remote_copy(
            src_ref=r_ref,
            dst_ref=yrecv_ref,
            send_sem=send_sems.at[1],
            recv_sem=recv_sems.at[1],
            device_id=(px, 1 - py),
            device_id_type=pl.DeviceIdType.MESH,
        )
        y_rdma.start()
        y_rdma.wait()

        @pl.when(py == 0)
        def _():
            out_ref[:, :fh] = r_ref[...]
            out_ref[:, fh:] = yrecv_ref[...]

        @pl.when(py == 1)
        def _():
            out_ref[:, fh:] = r_ref[...]
            out_ref[:, :fh] = yrecv_ref[...]

    return pl.pallas_call(
        body,
        out_shape=jax.ShapeDtypeStruct((dh, f), jnp.float32),
        in_specs=[pl.BlockSpec(memory_space=pltpu.VMEM),
                  pl.BlockSpec(memory_space=pltpu.VMEM)],
        out_specs=pl.BlockSpec(memory_space=pltpu.VMEM),
        scratch_shapes=[
            pltpu.VMEM((d, fh), jnp.float32),
            pltpu.VMEM((dh, fh), jnp.float32),
            pltpu.VMEM((dh, fh), jnp.float32),
            pltpu.VMEM((dh, fh), jnp.float32),
            pltpu.SemaphoreType.DMA((2,)),
            pltpu.SemaphoreType.DMA((2,)),
        ],
        compiler_params=pltpu.CompilerParams(collective_id=0),
    )(x, dy)


# device time: 27145 ns/iter; 1.2698x vs baseline; 1.2698x over previous
import jax
import jax.numpy as jnp
from jax import lax
from jax.experimental import pallas as pl
from jax.experimental.pallas import tpu as pltpu

C = 8


def kernel(x, dy):
    m, d = x.shape
    _, f = dy.shape
    fh = f // 2
    dh = d // 2
    cw = fh // C

    tdims = (((0,), (0,)), ((), ()))

    def body(x_ref, dy_ref, out_ref, p_ref, xrecv_ref,
             sx, rx, sy, ry):
        px = lax.axis_index("x")
        py = lax.axis_index("y")

        barrier = pltpu.get_barrier_semaphore()
        pl.semaphore_signal(barrier, inc=1, device_id=(1 - px, py),
                            device_id_type=pl.DeviceIdType.MESH)
        pl.semaphore_signal(barrier, inc=1, device_id=(px, 1 - py),
                            device_id_type=pl.DeviceIdType.MESH)
        pl.semaphore_wait(barrier, 2)

        def run(col0):
            oc0 = fh - col0
            x_rdmas = []
            y_rdmas = []

            def process(c):
                lo = c * cw
                x_rdmas[c].wait_recv()
                out_ref[:, col0 + lo:col0 + lo + cw] = (
                    p_ref[pl.ds(px * dh, dh), lo:lo + cw]
                    + xrecv_ref[:, lo:lo + cw])
                y_rdma = pltpu.make_async_remote_copy(
                    src_ref=out_ref.at[:, col0 + lo:col0 + lo + cw],
                    dst_ref=out_ref.at[:, col0 + lo:col0 + lo + cw],
                    send_sem=sy.at[c],
                    recv_sem=ry.at[c],
                    device_id=(px, 1 - py),
                    device_id_type=pl.DeviceIdType.MESH,
                )
                y_rdma.start()
                y_rdmas.append(y_rdma)

            for c in range(C):
                lo = c * cw
                p_ref[:, lo:lo + cw] = lax.dot_general(
                    x_ref[...], dy_ref[:, col0 + lo:col0 + lo + cw], tdims,
                    preferred_element_type=jnp.float32)
                x_rdma = pltpu.make_async_remote_copy(
                    src_ref=p_ref.at[pl.ds((1 - px) * dh, dh), lo:lo + cw],
                    dst_ref=xrecv_ref.at[:, lo:lo + cw],
                    send_sem=sx.at[c],
                    recv_sem=rx.at[c],
                    device_id=(1 - px, py),
                    device_id_type=pl.DeviceIdType.MESH,
                )
                x_rdma.start()
                x_rdmas.append(x_rdma)
                if c >= 1:
                    process(c - 1)
            process(C - 1)

            for c in range(C):
                lo = c * cw
                yin = pltpu.make_async_remote_copy(
                    src_ref=out_ref.at[:, oc0 + lo:oc0 + lo + cw],
                    dst_ref=out_ref.at[:, oc0 + lo:oc0 + lo + cw],
                    send_sem=sy.at[c],
                    recv_sem=ry.at[c],
                    device_id=(px, 1 - py),
                    device_id_type=pl.DeviceIdType.MESH,
                )
                yin.wait_recv()

            for r in x_rdmas:
                r.wait_send()
            for r in y_rdmas:
                r.wait_send()

        pl.when(py == 0)(lambda: run(0))
        pl.when(py == 1)(lambda: run(fh))

    return pl.pallas_call(
        body,
        out_shape=jax.ShapeDtypeStruct((dh, f), jnp.float32),
        in_specs=[pl.BlockSpec(memory_space=pltpu.VMEM),
                  pl.BlockSpec(memory_space=pltpu.VMEM)],
        out_specs=pl.BlockSpec(memory_space=pltpu.VMEM),
        scratch_shapes=[
            pltpu.VMEM((d, fh), jnp.float32),
            pltpu.VMEM((dh, fh), jnp.float32),
            pltpu.SemaphoreType.DMA((C,)),
            pltpu.SemaphoreType.DMA((C,)),
            pltpu.SemaphoreType.DMA((C,)),
            pltpu.SemaphoreType.DMA((C,)),
        ],
        compiler_params=pltpu.CompilerParams(collective_id=0),
    )(x, dy)


# device time: 27075 ns/iter; 1.2731x vs baseline; 1.0026x over previous
import jax
import jax.numpy as jnp
from jax import lax
from jax.experimental import pallas as pl
from jax.experimental.pallas import tpu as pltpu

C = 8


def kernel(x, dy):
    m, d = x.shape
    _, f = dy.shape
    fh = f // 2
    dh = d // 2
    cw = fh // C

    tdims = (((0,), (0,)), ((), ()))

    def body(x_ref, dy_ref, out_ref, p_ref, xr_ref, r_ref, yr_ref,
             sx, rx, sy, ry):
        px = lax.axis_index("x")
        py = lax.axis_index("y")

        barrier = pltpu.get_barrier_semaphore()
        pl.semaphore_signal(barrier, inc=1, device_id=(1 - px, py),
                            device_id_type=pl.DeviceIdType.MESH)
        pl.semaphore_signal(barrier, inc=1, device_id=(px, 1 - py),
                            device_id_type=pl.DeviceIdType.MESH)
        pl.semaphore_wait(barrier, 2)

        def run(col0):
            oc0 = fh - col0
            x_rdmas = []
            y_rdmas = []

            def process(c):
                lo = c * cw
                x_rdmas[c].wait_recv()
                red = p_ref[c, pl.ds(px * dh, dh), :] + xr_ref[c]
                r_ref[c] = red
                out_ref[:, col0 + lo:col0 + lo + cw] = red
                y_rdma = pltpu.make_async_remote_copy(
                    src_ref=r_ref.at[c],
                    dst_ref=yr_ref.at[c],
                    send_sem=sy.at[c],
                    recv_sem=ry.at[c],
                    device_id=(px, 1 - py),
                    device_id_type=pl.DeviceIdType.MESH,
                )
                y_rdma.start()
                y_rdmas.append(y_rdma)

            for c in range(C):
                lo = c * cw
                p_ref[c] = lax.dot_general(
                    x_ref[...], dy_ref[:, col0 + lo:col0 + lo + cw], tdims,
                    preferred_element_type=jnp.float32)
                x_rdma = pltpu.make_async_remote_copy(
                    src_ref=p_ref.at[c].at[pl.ds((1 - px) * dh, dh), :],
                    dst_ref=xr_ref.at[c],
                    send_sem=sx.at[c],
                    recv_sem=rx.at[c],
                    device_id=(1 - px, py),
                    device_id_type=pl.DeviceIdType.MESH,
                )
                x_rdma.start()
                x_rdmas.append(x_rdma)
                if c >= 1:
                    process(c - 1)
            process(C - 1)

            for c in range(C):
                lo = c * cw
                yin = pltpu.make_async_remote_copy(
                    src_ref=r_ref.at[c],
                    dst_ref=yr_ref.at[c],
                    send_sem=sy.at[c],
                    recv_sem=ry.at[c],
                    device_id=(px, 1 - py),
                    device_id_type=pl.DeviceIdType.MESH,
                )
                yin.wait_recv()
                out_ref[:, oc0 + lo:oc0 + lo + cw] = yr_ref[c]

            for rr in x_rdmas:
                rr.wait_send()
            for rr in y_rdmas:
                rr.wait_send()

        pl.when(py == 0)(lambda: run(0))
        pl.when(py == 1)(lambda: run(fh))

    return pl.pallas_call(
        body,
        out_shape=jax.ShapeDtypeStruct((dh, f), jnp.float32),
        in_specs=[pl.BlockSpec(memory_space=pltpu.VMEM),
                  pl.BlockSpec(memory_space=pltpu.VMEM)],
        out_specs=pl.BlockSpec(memory_space=pltpu.VMEM),
        scratch_shapes=[
            pltpu.VMEM((C, d, cw), jnp.float32),
            pltpu.VMEM((C, dh, cw), jnp.float32),
            pltpu.VMEM((C, dh, cw), jnp.float32),
            pltpu.VMEM((C, dh, cw), jnp.float32),
            pltpu.SemaphoreType.DMA((C,)),
            pltpu.SemaphoreType.DMA((C,)),
            pltpu.SemaphoreType.DMA((C,)),
            pltpu.SemaphoreType.DMA((C,)),
        ],
        compiler_params=pltpu.CompilerParams(collective_id=0),
    )(x, dy)
